# Initial kernel scaffold; baseline (speedup 1.0000x reference)
import jax
import jax.numpy as jnp
from jax.experimental import pallas as pl


def kernel(xyz, atom_xyz, atomtypes, batch, atom_batch, tt_W1, tt_b1, tt_W2, tt_b2, aa_W1, aa_b1, aa_W2, aa_b2, aa_g, aa_beta, em_W1, em_b1, em_W2, em_b2, em_g, em_beta):
    raise NotImplementedError("write your pallas kernel here")


# speedup vs baseline: 7.7185x; 7.7185x over previous
import functools

import jax
import jax.numpy as jnp
from jax import lax
from jax.experimental import pallas as pl
from jax.experimental.pallas import tpu as pltpu
from jax.experimental.pallas import tpu_sc as plsc

D = 32
K = 16
H = 2 * D + 1
NA = 5000
NS = 10000
NA_PAD = 5120
NS_PAD = 10240
BM = 256
BMP = 512
BIGF = 3.0e38
BIGI = 2**30
MASKED = 1e10


def _leaky(x):
    return jnp.where(x >= 0, x, 0.2 * x)


def _typemlp_body(at_ref, w1_ref, b1_ref, w2_ref, b2_ref, out_ref):
    h = _leaky(jnp.dot(at_ref[...], w1_ref[...],
                       preferred_element_type=jnp.float32) + b1_ref[...])
    out_ref[...] = jnp.dot(h, w2_ref[...],
                           preferred_element_type=jnp.float32) + b2_ref[...]


def _typemlp(at, w1, b1, w2, b2):
    return pl.pallas_call(
        _typemlp_body,
        out_shape=jax.ShapeDtypeStruct((NA, D), jnp.float32),
    )(at, w1, b1.reshape(1, D), w2, b2.reshape(1, D))


IDX_BITS = 0x1FFF


CW = 512


def _knn_body(x0, x1, x2, xb, y0, y1, y2, yb, idx_ref, dv_ref,
              key_s, *, skip_self, bm, nyp):
    i = pl.program_id(0)
    xbv = xb[...]
    ybv = yb[...]
    colr = lax.broadcasted_iota(jnp.int32, (1, nyp), 1)
    maxi = jnp.int32(0x7FFFFFFF)

    bmin = jnp.min(jnp.where(xbv >= 0, xbv, 99))
    bmax = jnp.max(jnp.where(xbv >= 0, xbv, -1))
    pos_first = jnp.min(jnp.where(ybv == bmin, colr, maxi))
    pos_last = jnp.max(jnp.where(ybv == bmax, colr, -1))
    c_lo = pos_first // CW
    c_hi = (pos_last + CW) // CW

    def build(c, carry):
        sl = pl.ds(c * CW, CW)
        dx = x0[...] - y0[:, sl]
        dy = x1[...] - y1[:, sl]
        dz = x2[...] - y2[:, sl]
        d2 = dx * dx + dy * dy + dz * dz
        d2m = jnp.where(xbv == yb[:, sl], d2, MASKED)
        colc = lax.broadcasted_iota(jnp.int32, (bm, CW), 1) + c * CW
        if skip_self:
            rowg = lax.broadcasted_iota(jnp.int32, (bm, CW), 0) + i * bm
            d2m = jnp.where(colc == rowg, BIGF, d2m)
        key_s[:, sl] = (lax.bitcast_convert_type(d2m, jnp.int32)
                        & jnp.int32(~IDX_BITS)) | colc
        return carry

    lax.fori_loop(c_lo, c_hi, build, 0)

    kcol = lax.broadcasted_iota(jnp.int32, (bm, K), 1)
    idxacc = jnp.zeros((bm, K), jnp.int32)
    dacc = jnp.zeros((bm, K), jnp.float32)
    last = jnp.full((bm, 1), -1, jnp.int32)
    for kk in range(K):
        def scan(c, acc):
            keys = key_s[:, pl.ds(c * CW, CW)]
            cm = jnp.min(jnp.where(keys > last, keys, maxi),
                         axis=1, keepdims=True)
            return jnp.minimum(acc, cm)

        kmin = lax.fori_loop(c_lo, c_hi, scan,
                             jnp.full((bm, 1), maxi, jnp.int32))
        jidx = jnp.where(kmin == maxi, 0, kmin & IDX_BITS)
        idxacc = jnp.where(kcol == kk, jidx, idxacc)
        val = lax.bitcast_convert_type(kmin & jnp.int32(~IDX_BITS),
                                       jnp.float32)
        dacc = jnp.where(kcol == kk, val, dacc)
        last = kmin
    idx_ref[...] = idxacc
    dv_ref[...] = dacc


def _knn(xc, xb_col, yc, yb_row, nxp, skip_self):
    nyp = NA_PAD
    grid = nxp // BM
    xspec = pl.BlockSpec((BM, 1), lambda i: (i, 0))
    yspec = pl.BlockSpec((1, nyp), lambda i: (0, 0))
    body = functools.partial(_knn_body, skip_self=skip_self, bm=BM, nyp=nyp)
    return pl.pallas_call(
        body,
        grid=(grid,),
        in_specs=[xspec, xspec, xspec, xspec, yspec, yspec, yspec, yspec],
        out_specs=[pl.BlockSpec((BM, K), lambda i: (i, 0)),
                   pl.BlockSpec((BM, K), lambda i: (i, 0))],
        out_shape=[jax.ShapeDtypeStruct((nxp, K), jnp.int32),
                   jax.ShapeDtypeStruct((nxp, K), jnp.float32)],
        scratch_shapes=[pltpu.VMEM((BM, nyp), jnp.int32)],
    )(xc[:, 0:1], xc[:, 1:2], xc[:, 2:3], xb_col,
      yc[:, 0:1].T, yc[:, 1:2].T, yc[:, 2:3].T, yb_row)


def _make_sc_gather(n_edges):
    info = plsc.get_sparse_core_info()
    nw = info.num_cores * info.num_subcores
    b_per_w = n_edges // nw
    nch = b_per_w // 128

    mesh = plsc.VectorSubcoreMesh(core_axis_name="c", subcore_axis_name="s")

    @functools.partial(
        pl.kernel, mesh=mesh,
        compiler_params=pltpu.CompilerParams(use_tc_tiling_on_sc=False),
        out_type=jax.ShapeDtypeStruct((n_edges, D), jnp.float32),
        scratch_types=[pltpu.VMEM((b_per_w,), jnp.int32),
                       pltpu.VMEM((128, D), jnp.float32),
                       pltpu.VMEM((128, D), jnp.float32),
                       pltpu.SemaphoreType.DMA,
                       pltpu.SemaphoreType.DMA,
                       pltpu.SemaphoreType.DMA,
                       pltpu.SemaphoreType.DMA],
    )
    def gather_k(table_hbm, idx_hbm, out_hbm, idx_v, rows0, rows1,
                 g0, g1, w0, w1):
        wid = lax.axis_index("s") * info.num_cores + lax.axis_index("c")
        base = wid * b_per_w
        pltpu.sync_copy(idx_hbm.at[pl.ds(base, b_per_w)], idx_v)

        def body(c, carry):
            o = c * 256
            ga = pltpu.async_copy(
                table_hbm.at[idx_v.at[pl.ds(o, 128)]], rows0, g0)
            gb = pltpu.async_copy(
                table_hbm.at[idx_v.at[pl.ds(o + 128, 128)]], rows1, g1)
            ga.wait()
            wa = pltpu.async_copy(
                rows0, out_hbm.at[pl.ds(base + o, 128)], w0)
            gb.wait()
            wb = pltpu.async_copy(
                rows1, out_hbm.at[pl.ds(base + o + 128, 128)], w1)
            wa.wait()
            wb.wait()
            return carry

        lax.fori_loop(0, nch // 2, body, 0)

    return gather_k


def _mp_body(self_ref, gath_ref, dist_ref, w1t, w1m, w1l, b1, w2, b2,
             g_ref, bt_ref, out_ref, *, bm):
    gath = gath_ref[...]
    selfF = self_ref[...]
    G = jnp.dot(gath, w1m[...], preferred_element_type=jnp.float32)
    S = jnp.dot(selfF, w1t[...], preferred_element_type=jnp.float32)
    DW = dist_ref[...] * w1l[...]
    pre = G + DW + b1[...]
    pre3 = pre.reshape(bm, K, H) + S[:, None, :]
    hs = jnp.sum(_leaky(pre3), axis=1)
    msg = jnp.dot(hs, w2[...], preferred_element_type=jnp.float32) \
        + jnp.float32(K) * b2[...]

    col = lax.broadcasted_iota(jnp.int32, (bm, D), 1)
    mA = col < (D // 2)
    half = jnp.float32(D // 2)
    sA = jnp.sum(jnp.where(mA, msg, 0.0), axis=1, keepdims=True) / half
    sB = jnp.sum(jnp.where(mA, 0.0, msg), axis=1, keepdims=True) / half
    mean = jnp.where(mA, sA, sB)
    dm = msg - mean
    vA = jnp.sum(jnp.where(mA, dm * dm, 0.0), axis=1, keepdims=True) / half
    vB = jnp.sum(jnp.where(mA, 0.0, dm * dm), axis=1, keepdims=True) / half
    var = jnp.where(mA, vA, vB)
    xn = dm / jnp.sqrt(var + 1e-5)
    gn = xn * g_ref[...] + bt_ref[...]
    out_ref[...] = selfF + _leaky(gn)


def _mp_layer(self_feats, gath, dist_edge, w1, b1, w2, b2, g, bt, n_pad):
    grid = n_pad // BMP
    body = functools.partial(_mp_body, bm=BMP)
    wspec = pl.BlockSpec(None, lambda i: (0, 0))
    return pl.pallas_call(
        body,
        grid=(grid,),
        in_specs=[pl.BlockSpec((BMP, D), lambda i: (i, 0)),
                  pl.BlockSpec((BMP * K, D), lambda i: (i, 0)),
                  pl.BlockSpec((BMP * K, 1), lambda i: (i, 0)),
                  wspec, wspec, wspec, wspec, wspec, wspec, wspec, wspec],
        out_specs=pl.BlockSpec((BMP, D), lambda i: (i, 0)),
        out_shape=jax.ShapeDtypeStruct((n_pad, D), jnp.float32),
    )(self_feats, gath, dist_edge,
      w1[:D, :], w1[D:2 * D, :], w1[2 * D:, :], b1.reshape(1, H),
      w2, b2.reshape(1, D), g.reshape(1, D), bt.reshape(1, D))


def kernel(xyz, atom_xyz, atomtypes, batch, atom_batch, tt_W1, tt_b1,
           tt_W2, tt_b2, aa_W1, aa_b1, aa_W2, aa_b2, aa_g, aa_beta,
           em_W1, em_b1, em_W2, em_b2, em_g, em_beta):
    f32 = jnp.float32
    axyz_p = jnp.pad(atom_xyz, ((0, NA_PAD - NA), (0, 0)))
    xyz_p = jnp.pad(xyz, ((0, NS_PAD - NS), (0, 0)))
    ab = atom_batch.astype(jnp.int32)
    sb = batch.astype(jnp.int32)
    ab_q = jnp.pad(ab, (0, NA_PAD - NA), constant_values=-2).reshape(-1, 1)
    ab_k = jnp.pad(ab, (0, NA_PAD - NA), constant_values=-1).reshape(1, -1)
    sb_q = jnp.pad(sb, (0, NS_PAD - NS), constant_values=-2).reshape(-1, 1)

    t = _typemlp(atomtypes, tt_W1, tt_b1, tt_W2, tt_b2)
    feats = jnp.pad(t, ((0, NA_PAD - NA), (0, 0)))

    idx_aa, d_aa = _knn(axyz_p, ab_q, axyz_p, ab_k, NA_PAD, True)
    idx_aa_f = idx_aa.reshape(-1)
    d_aa_e = d_aa.reshape(-1, 1).astype(f32)

    gather_aa = _make_sc_gather(NA_PAD * K)
    for i in range(3):
        gath = gather_aa(feats, idx_aa_f)
        feats = _mp_layer(feats, gath, d_aa_e, aa_W1[i], aa_b1[i],
                          aa_W2[i], aa_b2[i], aa_g[i], aa_beta[i], NA_PAD)

    idx_em, d_em = _knn(xyz_p, sb_q, axyz_p, ab_k, NS_PAD, False)
    idx_em_f = idx_em.reshape(-1)
    d_em_e = d_em.reshape(-1, 1).astype(f32)

    gath_em = _make_sc_gather(NS_PAD * K)(feats, idx_em_f)
    pe = jnp.ones((NS_PAD, D), f32)
    for i in range(3):
        pe = _mp_layer(pe, gath_em, d_em_e, em_W1[i], em_b1[i],
                       em_W2[i], em_b2[i], em_g[i], em_beta[i], NS_PAD)
    return pe[:NS]


# speedup vs baseline: 8.0150x; 1.0384x over previous
import functools

import jax
import jax.numpy as jnp
from jax import lax
from jax.experimental import pallas as pl
from jax.experimental.pallas import tpu as pltpu
from jax.experimental.pallas import tpu_sc as plsc

D = 32
K = 16
H = 2 * D + 1
NA = 5000
NS = 10000
NA_PAD = 5120
NS_PAD = 10240
BM = 256
BMP = 512
BIGF = 3.0e38
BIGI = 2**30
MASKED = 1e10


def _leaky(x):
    return jnp.where(x >= 0, x, 0.2 * x)


def _typemlp_body(at_ref, w1_ref, b1_ref, w2_ref, b2_ref, out_ref):
    h = _leaky(jnp.dot(at_ref[...], w1_ref[...],
                       preferred_element_type=jnp.float32) + b1_ref[...])
    out_ref[...] = jnp.dot(h, w2_ref[...],
                           preferred_element_type=jnp.float32) + b2_ref[...]


def _typemlp(at, w1, b1, w2, b2):
    return pl.pallas_call(
        _typemlp_body,
        out_shape=jax.ShapeDtypeStruct((NA, D), jnp.float32),
    )(at, w1, b1.reshape(1, D), w2, b2.reshape(1, D))


IDX_BITS = 0x1FFF


CW = 512


def _knn_body(x0, x1, x2, xb, y0, y1, y2, yb, idx_ref, dv_ref,
              key_s, *, skip_self, bm, nyp):
    i = pl.program_id(0)
    xbv = xb[...]
    ybv = yb[...]
    colr = lax.broadcasted_iota(jnp.int32, (1, nyp), 1)
    maxi = jnp.int32(0x7FFFFFFF)

    bmin = jnp.min(jnp.where(xbv >= 0, xbv, 99))
    bmax = jnp.max(jnp.where(xbv >= 0, xbv, -1))
    pos_first = jnp.min(jnp.where(ybv == bmin, colr, maxi))
    pos_last = jnp.max(jnp.where(ybv == bmax, colr, -1))
    c_lo = pos_first // CW
    c_hi = (pos_last + CW) // CW

    def build(c, carry):
        sl = pl.ds(c * CW, CW)
        dx = x0[...] - y0[:, sl]
        dy = x1[...] - y1[:, sl]
        dz = x2[...] - y2[:, sl]
        d2 = dx * dx + dy * dy + dz * dz
        d2m = jnp.where(xbv == yb[:, sl], d2, MASKED)
        colc = lax.broadcasted_iota(jnp.int32, (bm, CW), 1) + c * CW
        if skip_self:
            rowg = lax.broadcasted_iota(jnp.int32, (bm, CW), 0) + i * bm
            d2m = jnp.where(colc == rowg, BIGF, d2m)
        key_s[:, sl] = (lax.bitcast_convert_type(d2m, jnp.int32)
                        & jnp.int32(~IDX_BITS)) | colc
        return carry

    lax.fori_loop(c_lo, c_hi, build, 0)

    kcol = lax.broadcasted_iota(jnp.int32, (bm, K), 1)
    idxacc = jnp.zeros((bm, K), jnp.int32)
    dacc = jnp.zeros((bm, K), jnp.float32)
    last = jnp.full((bm, 1), -1, jnp.int32)
    for kk in range(K):
        def scan(c, acc):
            keys = key_s[:, pl.ds(c * CW, CW)]
            cm = jnp.min(jnp.where(keys > last, keys, maxi),
                         axis=1, keepdims=True)
            return jnp.minimum(acc, cm)

        kmin = lax.fori_loop(c_lo, c_hi, scan,
                             jnp.full((bm, 1), maxi, jnp.int32))
        jidx = jnp.where(kmin == maxi, 0, kmin & IDX_BITS)
        idxacc = jnp.where(kcol == kk, jidx, idxacc)
        val = lax.bitcast_convert_type(kmin & jnp.int32(~IDX_BITS),
                                       jnp.float32)
        dacc = jnp.where(kcol == kk, val, dacc)
        last = kmin
    idx_ref[...] = idxacc
    dv_ref[...] = dacc


def _knn(xc, xb_col, yc, yb_row, nxp, skip_self):
    nyp = NA_PAD
    grid = nxp // BM
    xspec = pl.BlockSpec((BM, 1), lambda i: (i, 0))
    yspec = pl.BlockSpec((1, nyp), lambda i: (0, 0))
    body = functools.partial(_knn_body, skip_self=skip_self, bm=BM, nyp=nyp)
    return pl.pallas_call(
        body,
        grid=(grid,),
        in_specs=[xspec, xspec, xspec, xspec, yspec, yspec, yspec, yspec],
        out_specs=[pl.BlockSpec((BM, K), lambda i: (i, 0)),
                   pl.BlockSpec((BM, K), lambda i: (i, 0))],
        out_shape=[jax.ShapeDtypeStruct((nxp, K), jnp.int32),
                   jax.ShapeDtypeStruct((nxp, K), jnp.float32)],
        scratch_shapes=[pltpu.VMEM((BM, nyp), jnp.int32)],
    )(xc[:, 0:1], xc[:, 1:2], xc[:, 2:3], xb_col,
      yc[:, 0:1].T, yc[:, 1:2].T, yc[:, 2:3].T, yb_row)


def _make_sc_gather(n_edges):
    info = plsc.get_sparse_core_info()
    nw = info.num_cores * info.num_subcores
    b_per_w = n_edges // nw
    nch = b_per_w // 128

    mesh = plsc.VectorSubcoreMesh(core_axis_name="c", subcore_axis_name="s")

    @functools.partial(
        pl.kernel, mesh=mesh,
        compiler_params=pltpu.CompilerParams(use_tc_tiling_on_sc=False),
        out_type=jax.ShapeDtypeStruct((n_edges, D), jnp.float32),
        scratch_types=[pltpu.VMEM((b_per_w,), jnp.int32),
                       pltpu.VMEM((128, D), jnp.float32),
                       pltpu.VMEM((128, D), jnp.float32),
                       pltpu.SemaphoreType.DMA,
                       pltpu.SemaphoreType.DMA,
                       pltpu.SemaphoreType.DMA,
                       pltpu.SemaphoreType.DMA],
    )
    def gather_k(table_hbm, idx_hbm, out_hbm, idx_v, rows0, rows1,
                 g0, g1, w0, w1):
        wid = lax.axis_index("s") * info.num_cores + lax.axis_index("c")
        base = wid * b_per_w
        pltpu.sync_copy(idx_hbm.at[pl.ds(base, b_per_w)], idx_v)

        def body(c, carry):
            o = c * 256
            ga = pltpu.async_copy(
                table_hbm.at[idx_v.at[pl.ds(o, 128)]], rows0, g0)
            gb = pltpu.async_copy(
                table_hbm.at[idx_v.at[pl.ds(o + 128, 128)]], rows1, g1)
            ga.wait()
            wa = pltpu.async_copy(
                rows0, out_hbm.at[pl.ds(base + o, 128)], w0)
            gb.wait()
            wb = pltpu.async_copy(
                rows1, out_hbm.at[pl.ds(base + o + 128, 128)], w1)
            wa.wait()
            wb.wait()
            return carry

        lax.fori_loop(0, nch // 2, body, 0)

    return gather_k


def _mp_body(self_ref, gath_ref, dist_ref, w1t, w1m, w1l, b1, w2, b2,
             g_ref, bt_ref, out_ref, *, bm):
    gath = gath_ref[...]
    selfF = self_ref[...]
    G = jnp.dot(gath, w1m[...], preferred_element_type=jnp.float32)
    S = jnp.dot(selfF, w1t[...], preferred_element_type=jnp.float32)
    DW = dist_ref[...] * w1l[...]
    pre = G + DW + b1[...]
    pre3 = pre.reshape(bm, K, H) + S[:, None, :]
    hs = jnp.sum(_leaky(pre3), axis=1)
    msg = jnp.dot(hs, w2[...], preferred_element_type=jnp.float32) \
        + jnp.float32(K) * b2[...]

    col = lax.broadcasted_iota(jnp.int32, (bm, D), 1)
    mA = col < (D // 2)
    half = jnp.float32(D // 2)
    sA = jnp.sum(jnp.where(mA, msg, 0.0), axis=1, keepdims=True) / half
    sB = jnp.sum(jnp.where(mA, 0.0, msg), axis=1, keepdims=True) / half
    mean = jnp.where(mA, sA, sB)
    dm = msg - mean
    vA = jnp.sum(jnp.where(mA, dm * dm, 0.0), axis=1, keepdims=True) / half
    vB = jnp.sum(jnp.where(mA, 0.0, dm * dm), axis=1, keepdims=True) / half
    var = jnp.where(mA, vA, vB)
    xn = dm / jnp.sqrt(var + 1e-5)
    gn = xn * g_ref[...] + bt_ref[...]
    out_ref[...] = selfF + _leaky(gn)


def _mp_layer(self_feats, gath, dist_edge, w1, b1, w2, b2, g, bt, n_pad):
    grid = n_pad // BMP
    body = functools.partial(_mp_body, bm=BMP)
    wspec = pl.BlockSpec(None, lambda i: (0, 0))
    return pl.pallas_call(
        body,
        grid=(grid,),
        in_specs=[pl.BlockSpec((BMP, D), lambda i: (i, 0)),
                  pl.BlockSpec((BMP * K, D), lambda i: (i, 0)),
                  pl.BlockSpec((BMP * K, 1), lambda i: (i, 0)),
                  wspec, wspec, wspec, wspec, wspec, wspec, wspec, wspec],
        out_specs=pl.BlockSpec((BMP, D), lambda i: (i, 0)),
        out_shape=jax.ShapeDtypeStruct((n_pad, D), jnp.float32),
    )(self_feats, gath, dist_edge,
      w1[:D, :], w1[D:2 * D, :], w1[2 * D:, :], b1.reshape(1, H),
      w2, b2.reshape(1, D), g.reshape(1, D), bt.reshape(1, D))


def _mp3_body(gath_ref, dist_ref, *rest, bm):
    wrefs, out_ref = rest[:-1], rest[-1]
    gath = gath_ref[...]
    dist = dist_ref[...]
    selfF = jnp.ones((bm, D), jnp.float32)
    col = lax.broadcasted_iota(jnp.int32, (bm, D), 1)
    mA = col < (D // 2)
    half = jnp.float32(D // 2)
    for li in range(3):
        w1t, w1m, w1l, b1, w2, b2, g, bt = wrefs[8 * li:8 * li + 8]
        G = jnp.dot(gath, w1m[...], preferred_element_type=jnp.float32)
        S = jnp.dot(selfF, w1t[...], preferred_element_type=jnp.float32)
        pre = G + dist * w1l[...] + b1[...]
        pre3 = pre.reshape(bm, K, H) + S[:, None, :]
        hs = jnp.sum(_leaky(pre3), axis=1)
        msg = jnp.dot(hs, w2[...], preferred_element_type=jnp.float32) \
            + jnp.float32(K) * b2[...]
        sA = jnp.sum(jnp.where(mA, msg, 0.0), axis=1, keepdims=True) / half
        sB = jnp.sum(jnp.where(mA, 0.0, msg), axis=1, keepdims=True) / half
        dm = msg - jnp.where(mA, sA, sB)
        vA = jnp.sum(jnp.where(mA, dm * dm, 0.0), axis=1,
                     keepdims=True) / half
        vB = jnp.sum(jnp.where(mA, 0.0, dm * dm), axis=1,
                     keepdims=True) / half
        xn = dm / jnp.sqrt(jnp.where(mA, vA, vB) + 1e-5)
        selfF = selfF + _leaky(xn * g[...] + bt[...])
    out_ref[...] = selfF


def _mp3(gath, dist_edge, W1, B1, W2, B2, Gm, Bt):
    grid = NS_PAD // BMP
    body = functools.partial(_mp3_body, bm=BMP)
    wspec = pl.BlockSpec(None, lambda i: (0, 0))
    wargs = []
    for i in range(3):
        wargs += [W1[i][:D, :], W1[i][D:2 * D, :], W1[i][2 * D:, :],
                  B1[i].reshape(1, H), W2[i], B2[i].reshape(1, D),
                  Gm[i].reshape(1, D), Bt[i].reshape(1, D)]
    return pl.pallas_call(
        body,
        grid=(grid,),
        in_specs=[pl.BlockSpec((BMP * K, D), lambda i: (i, 0)),
                  pl.BlockSpec((BMP * K, 1), lambda i: (i, 0))] +
                 [wspec] * 24,
        out_specs=pl.BlockSpec((BMP, D), lambda i: (i, 0)),
        out_shape=jax.ShapeDtypeStruct((NS_PAD, D), jnp.float32),
    )(gath, dist_edge, *wargs)


def kernel(xyz, atom_xyz, atomtypes, batch, atom_batch, tt_W1, tt_b1,
           tt_W2, tt_b2, aa_W1, aa_b1, aa_W2, aa_b2, aa_g, aa_beta,
           em_W1, em_b1, em_W2, em_b2, em_g, em_beta):
    f32 = jnp.float32
    axyz_p = jnp.pad(atom_xyz, ((0, NA_PAD - NA), (0, 0)))
    xyz_p = jnp.pad(xyz, ((0, NS_PAD - NS), (0, 0)))
    ab = atom_batch.astype(jnp.int32)
    sb = batch.astype(jnp.int32)
    ab_q = jnp.pad(ab, (0, NA_PAD - NA), constant_values=-2).reshape(-1, 1)
    ab_k = jnp.pad(ab, (0, NA_PAD - NA), constant_values=-1).reshape(1, -1)
    sb_q = jnp.pad(sb, (0, NS_PAD - NS), constant_values=-2).reshape(-1, 1)

    t = _typemlp(atomtypes, tt_W1, tt_b1, tt_W2, tt_b2)
    feats = jnp.pad(t, ((0, NA_PAD - NA), (0, 0)))

    idx_aa, d_aa = _knn(axyz_p, ab_q, axyz_p, ab_k, NA_PAD, True)
    idx_aa_f = idx_aa.reshape(-1)
    d_aa_e = d_aa.reshape(-1, 1).astype(f32)

    gather_aa = _make_sc_gather(NA_PAD * K)
    for i in range(3):
        gath = gather_aa(feats, idx_aa_f)
        feats = _mp_layer(feats, gath, d_aa_e, aa_W1[i], aa_b1[i],
                          aa_W2[i], aa_b2[i], aa_g[i], aa_beta[i], NA_PAD)

    idx_em, d_em = _knn(xyz_p, sb_q, axyz_p, ab_k, NS_PAD, False)
    idx_em_f = idx_em.reshape(-1)
    d_em_e = d_em.reshape(-1, 1).astype(f32)

    gath_em = _make_sc_gather(NS_PAD * K)(feats, idx_em_f)
    pe = _mp3(gath_em, d_em_e, em_W1, em_b1, em_W2, em_b2, em_g, em_beta)
    return pe[:NS]


# speedup vs baseline: 8.9607x; 1.1180x over previous
import functools

import jax
import jax.numpy as jnp
from jax import lax
from jax.experimental import pallas as pl
from jax.experimental.pallas import tpu as pltpu
from jax.experimental.pallas import tpu_sc as plsc

D = 32
K = 16
H = 2 * D + 1
NA = 5000
NS = 10000
NA_PAD = 5120
NS_PAD = 10240
BM = 256
BMP = 512
BIGF = 3.0e38
BIGI = 2**30
MASKED = 1e10


def _leaky(x):
    return jnp.where(x >= 0, x, 0.2 * x)


def _typemlp_body(at_ref, w1_ref, b1_ref, w2_ref, b2_ref, out_ref):
    h = _leaky(jnp.dot(at_ref[...], w1_ref[...],
                       preferred_element_type=jnp.float32) + b1_ref[...])
    out_ref[...] = jnp.dot(h, w2_ref[...],
                           preferred_element_type=jnp.float32) + b2_ref[...]


def _typemlp(at, w1, b1, w2, b2):
    return pl.pallas_call(
        _typemlp_body,
        out_shape=jax.ShapeDtypeStruct((NA, D), jnp.float32),
    )(at, w1, b1.reshape(1, D), w2, b2.reshape(1, D))


IDX_BITS = 0x1FFF


CW = 1024


def _knn_body(x0, x1, x2, xb, y0, y1, y2, yb, idx_ref, dv_ref,
              key_s, *, skip_self, bm, nyp):
    i = pl.program_id(0)
    xbv = xb[...]
    ybv = yb[...]
    colr = lax.broadcasted_iota(jnp.int32, (1, nyp), 1)
    maxi = jnp.int32(0x7FFFFFFF)

    bmin = jnp.min(jnp.where(xbv >= 0, xbv, 99))
    bmax = jnp.max(jnp.where(xbv >= 0, xbv, -1))
    pos_first = jnp.min(jnp.where(ybv == bmin, colr, maxi))
    pos_last = jnp.max(jnp.where(ybv == bmax, colr, -1))
    c_lo = pos_first // CW
    c_hi = (pos_last + CW) // CW

    def build(c, carry):
        sl = pl.ds(c * CW, CW)
        dx = x0[...] - y0[:, sl]
        dy = x1[...] - y1[:, sl]
        dz = x2[...] - y2[:, sl]
        d2 = dx * dx + dy * dy + dz * dz
        d2m = jnp.where(xbv == yb[:, sl], d2, MASKED)
        colc = lax.broadcasted_iota(jnp.int32, (bm, CW), 1) + c * CW
        if skip_self:
            rowg = lax.broadcasted_iota(jnp.int32, (bm, CW), 0) + i * bm
            d2m = jnp.where(colc == rowg, BIGF, d2m)
        key_s[:, sl] = (lax.bitcast_convert_type(d2m, jnp.int32)
                        & jnp.int32(~IDX_BITS)) | colc
        return carry

    lax.fori_loop(c_lo, c_hi, build, 0)

    kcol = lax.broadcasted_iota(jnp.int32, (bm, K), 1)
    idxacc = jnp.zeros((bm, K), jnp.int32)
    dacc = jnp.zeros((bm, K), jnp.float32)
    last = jnp.full((bm, 1), -1, jnp.int32)
    for kk in range(K):
        def scan(c, acc):
            keys = key_s[:, pl.ds(c * CW, CW)]
            cm = jnp.min(jnp.where(keys > last, keys, maxi),
                         axis=1, keepdims=True)
            return jnp.minimum(acc, cm)

        kmin = lax.fori_loop(c_lo, c_hi, scan,
                             jnp.full((bm, 1), maxi, jnp.int32))
        jidx = jnp.where(kmin == maxi, 0, kmin & IDX_BITS)
        idxacc = jnp.where(kcol == kk, jidx, idxacc)
        val = lax.bitcast_convert_type(kmin & jnp.int32(~IDX_BITS),
                                       jnp.float32)
        dacc = jnp.where(kcol == kk, val, dacc)
        last = kmin
    idx_ref[...] = idxacc
    dv_ref[...] = dacc


def _knn(xc, xb_col, yc, yb_row, nxp, skip_self):
    nyp = NA_PAD
    grid = nxp // BM
    xspec = pl.BlockSpec((BM, 1), lambda i: (i, 0))
    yspec = pl.BlockSpec((1, nyp), lambda i: (0, 0))
    body = functools.partial(_knn_body, skip_self=skip_self, bm=BM, nyp=nyp)
    return pl.pallas_call(
        body,
        grid=(grid,),
        in_specs=[xspec, xspec, xspec, xspec, yspec, yspec, yspec, yspec],
        out_specs=[pl.BlockSpec((BM, K), lambda i: (i, 0)),
                   pl.BlockSpec((BM, K), lambda i: (i, 0))],
        out_shape=[jax.ShapeDtypeStruct((nxp, K), jnp.int32),
                   jax.ShapeDtypeStruct((nxp, K), jnp.float32)],
        scratch_shapes=[pltpu.VMEM((BM, nyp), jnp.int32)],
    )(xc[:, 0:1], xc[:, 1:2], xc[:, 2:3], xb_col,
      yc[:, 0:1].T, yc[:, 1:2].T, yc[:, 2:3].T, yb_row)


def _make_sc_gather(n_edges):
    info = plsc.get_sparse_core_info()
    nw = info.num_cores * info.num_subcores
    b_per_w = n_edges // nw
    nch = b_per_w // 128

    mesh = plsc.VectorSubcoreMesh(core_axis_name="c", subcore_axis_name="s")

    @functools.partial(
        pl.kernel, mesh=mesh,
        compiler_params=pltpu.CompilerParams(use_tc_tiling_on_sc=False),
        out_type=jax.ShapeDtypeStruct((n_edges, D), jnp.float32),
        scratch_types=[pltpu.VMEM((b_per_w,), jnp.int32),
                       pltpu.VMEM((128, D), jnp.float32),
                       pltpu.VMEM((128, D), jnp.float32),
                       pltpu.SemaphoreType.DMA,
                       pltpu.SemaphoreType.DMA,
                       pltpu.SemaphoreType.DMA,
                       pltpu.SemaphoreType.DMA],
    )
    def gather_k(table_hbm, idx_hbm, out_hbm, idx_v, rows0, rows1,
                 g0, g1, w0, w1):
        wid = lax.axis_index("s") * info.num_cores + lax.axis_index("c")
        base = wid * b_per_w
        pltpu.sync_copy(idx_hbm.at[pl.ds(base, b_per_w)], idx_v)

        def body(c, carry):
            o = c * 256
            ga = pltpu.async_copy(
                table_hbm.at[idx_v.at[pl.ds(o, 128)]], rows0, g0)
            gb = pltpu.async_copy(
                table_hbm.at[idx_v.at[pl.ds(o + 128, 128)]], rows1, g1)
            ga.wait()
            wa = pltpu.async_copy(
                rows0, out_hbm.at[pl.ds(base + o, 128)], w0)
            gb.wait()
            wb = pltpu.async_copy(
                rows1, out_hbm.at[pl.ds(base + o + 128, 128)], w1)
            wa.wait()
            wb.wait()
            return carry

        lax.fori_loop(0, nch // 2, body, 0)

    return gather_k


def _mp_body(self_ref, gath_ref, dist_ref, w1t, w1m, w1l, b1, w2, b2,
             g_ref, bt_ref, out_ref, *, bm):
    gath = gath_ref[...]
    selfF = self_ref[...]
    G = jnp.dot(gath, w1m[...], preferred_element_type=jnp.float32)
    S = jnp.dot(selfF, w1t[...], preferred_element_type=jnp.float32)
    DW = dist_ref[...] * w1l[...]
    pre = G + DW + b1[...]
    pre3 = pre.reshape(bm, K, H) + S[:, None, :]
    hs = jnp.sum(_leaky(pre3), axis=1)
    msg = jnp.dot(hs, w2[...], preferred_element_type=jnp.float32) \
        + jnp.float32(K) * b2[...]

    col = lax.broadcasted_iota(jnp.int32, (bm, D), 1)
    mA = col < (D // 2)
    half = jnp.float32(D // 2)
    sA = jnp.sum(jnp.where(mA, msg, 0.0), axis=1, keepdims=True) / half
    sB = jnp.sum(jnp.where(mA, 0.0, msg), axis=1, keepdims=True) / half
    mean = jnp.where(mA, sA, sB)
    dm = msg - mean
    vA = jnp.sum(jnp.where(mA, dm * dm, 0.0), axis=1, keepdims=True) / half
    vB = jnp.sum(jnp.where(mA, 0.0, dm * dm), axis=1, keepdims=True) / half
    var = jnp.where(mA, vA, vB)
    xn = dm / jnp.sqrt(var + 1e-5)
    gn = xn * g_ref[...] + bt_ref[...]
    out_ref[...] = selfF + _leaky(gn)


def _mp_layer(self_feats, gath, dist_edge, w1, b1, w2, b2, g, bt, n_pad):
    grid = n_pad // BMP
    body = functools.partial(_mp_body, bm=BMP)
    wspec = pl.BlockSpec(None, lambda i: (0, 0))
    return pl.pallas_call(
        body,
        grid=(grid,),
        in_specs=[pl.BlockSpec((BMP, D), lambda i: (i, 0)),
                  pl.BlockSpec((BMP * K, D), lambda i: (i, 0)),
                  pl.BlockSpec((BMP * K, 1), lambda i: (i, 0)),
                  wspec, wspec, wspec, wspec, wspec, wspec, wspec, wspec],
        out_specs=pl.BlockSpec((BMP, D), lambda i: (i, 0)),
        out_shape=jax.ShapeDtypeStruct((n_pad, D), jnp.float32),
    )(self_feats, gath, dist_edge,
      w1[:D, :], w1[D:2 * D, :], w1[2 * D:, :], b1.reshape(1, H),
      w2, b2.reshape(1, D), g.reshape(1, D), bt.reshape(1, D))


def _mp3_body(gath_ref, dist_ref, *rest, bm):
    wrefs, out_ref = rest[:-1], rest[-1]
    gath = gath_ref[...]
    dist = dist_ref[...]
    selfF = jnp.ones((bm, D), jnp.float32)
    col = lax.broadcasted_iota(jnp.int32, (bm, D), 1)
    mA = col < (D // 2)
    half = jnp.float32(D // 2)
    for li in range(3):
        w1t, w1m, w1l, b1, w2, b2, g, bt = wrefs[8 * li:8 * li + 8]
        G = jnp.dot(gath, w1m[...], preferred_element_type=jnp.float32)
        S = jnp.dot(selfF, w1t[...], preferred_element_type=jnp.float32)
        pre = G + dist * w1l[...] + b1[...]
        pre3 = pre.reshape(bm, K, H) + S[:, None, :]
        hs = jnp.sum(_leaky(pre3), axis=1)
        msg = jnp.dot(hs, w2[...], preferred_element_type=jnp.float32) \
            + jnp.float32(K) * b2[...]
        sA = jnp.sum(jnp.where(mA, msg, 0.0), axis=1, keepdims=True) / half
        sB = jnp.sum(jnp.where(mA, 0.0, msg), axis=1, keepdims=True) / half
        dm = msg - jnp.where(mA, sA, sB)
        vA = jnp.sum(jnp.where(mA, dm * dm, 0.0), axis=1,
                     keepdims=True) / half
        vB = jnp.sum(jnp.where(mA, 0.0, dm * dm), axis=1,
                     keepdims=True) / half
        xn = dm / jnp.sqrt(jnp.where(mA, vA, vB) + 1e-5)
        selfF = selfF + _leaky(xn * g[...] + bt[...])
    out_ref[...] = selfF


def _mp3(gath, dist_edge, W1, B1, W2, B2, Gm, Bt):
    grid = NS_PAD // BMP
    body = functools.partial(_mp3_body, bm=BMP)
    wspec = pl.BlockSpec(None, lambda i: (0, 0))
    wargs = []
    for i in range(3):
        wargs += [W1[i][:D, :], W1[i][D:2 * D, :], W1[i][2 * D:, :],
                  B1[i].reshape(1, H), W2[i], B2[i].reshape(1, D),
                  Gm[i].reshape(1, D), Bt[i].reshape(1, D)]
    return pl.pallas_call(
        body,
        grid=(grid,),
        in_specs=[pl.BlockSpec((BMP * K, D), lambda i: (i, 0)),
                  pl.BlockSpec((BMP * K, 1), lambda i: (i, 0))] +
                 [wspec] * 24,
        out_specs=pl.BlockSpec((BMP, D), lambda i: (i, 0)),
        out_shape=jax.ShapeDtypeStruct((NS_PAD, D), jnp.float32),
    )(gath, dist_edge, *wargs)


def kernel(xyz, atom_xyz, atomtypes, batch, atom_batch, tt_W1, tt_b1,
           tt_W2, tt_b2, aa_W1, aa_b1, aa_W2, aa_b2, aa_g, aa_beta,
           em_W1, em_b1, em_W2, em_b2, em_g, em_beta):
    f32 = jnp.float32
    axyz_p = jnp.pad(atom_xyz, ((0, NA_PAD - NA), (0, 0)))
    xyz_p = jnp.pad(xyz, ((0, NS_PAD - NS), (0, 0)))
    ab = atom_batch.astype(jnp.int32)
    sb = batch.astype(jnp.int32)
    ab_q = jnp.pad(ab, (0, NA_PAD - NA), constant_values=-2).reshape(-1, 1)
    ab_k = jnp.pad(ab, (0, NA_PAD - NA), constant_values=-1).reshape(1, -1)
    sb_q = jnp.pad(sb, (0, NS_PAD - NS), constant_values=-2).reshape(-1, 1)

    t = _typemlp(atomtypes, tt_W1, tt_b1, tt_W2, tt_b2)
    feats = jnp.pad(t, ((0, NA_PAD - NA), (0, 0)))

    idx_aa, d_aa = _knn(axyz_p, ab_q, axyz_p, ab_k, NA_PAD, True)
    idx_aa_f = idx_aa.reshape(-1)
    d_aa_e = d_aa.reshape(-1, 1).astype(f32)

    gather_aa = _make_sc_gather(NA_PAD * K)
    for i in range(3):
        gath = gather_aa(feats, idx_aa_f)
        feats = _mp_layer(feats, gath, d_aa_e, aa_W1[i], aa_b1[i],
                          aa_W2[i], aa_b2[i], aa_g[i], aa_beta[i], NA_PAD)

    idx_em, d_em = _knn(xyz_p, sb_q, axyz_p, ab_k, NS_PAD, False)
    idx_em_f = idx_em.reshape(-1)
    d_em_e = d_em.reshape(-1, 1).astype(f32)

    gath_em = _make_sc_gather(NS_PAD * K)(feats, idx_em_f)
    pe = _mp3(gath_em, d_em_e, em_W1, em_b1, em_W2, em_b2, em_g, em_beta)
    return pe[:NS]
